# Initial kernel scaffold; baseline (speedup 1.0000x reference)
import jax
import jax.numpy as jnp
from jax.experimental import pallas as pl


def kernel(x, weight, bias, gamma, beta):
    raise NotImplementedError("write your pallas kernel here")


# speedup vs baseline: 1.2308x; 1.2308x over previous
import functools

import jax
import jax.numpy as jnp
from jax.experimental import pallas as pl
from jax.experimental.pallas import tpu as pltpu

_LANE = 128


def _round_up(x, m):
    return ((x + m - 1) // m) * m


def _conv_one(xp_b, w_flat, kernel_size, dilation, tl):
    taps = [
        jax.lax.slice_in_dim(xp_b, k * dilation, k * dilation + tl, axis=1)
        for k in range(kernel_size)
    ]
    xs = jnp.concatenate(taps, axis=0)
    return jax.lax.dot_general(
        w_flat, xs,
        dimension_numbers=(((1,), (0,)), ((), ())),
        preferred_element_type=jnp.float32)


def _stats_kernel(x_ref, w_ref, stats_ref, *, kernel_size, dilation, tl,
                  n_blk, l_out, c_out):
    s1 = jnp.zeros((c_out, 1), jnp.float32)
    s2 = jnp.zeros((c_out, 1), jnp.float32)
    w_flat = w_ref[...]
    for b in range(n_blk):
        conv = _conv_one(x_ref[b], w_flat, kernel_size, dilation, tl)
        if l_out != tl:
            lane = jax.lax.broadcasted_iota(jnp.int32, conv.shape, 1)
            conv = jnp.where(lane < l_out, conv, 0.0)
        s1 = s1 + jnp.sum(conv, axis=1, keepdims=True)
        s2 = s2 + jnp.sum(conv * conv, axis=1, keepdims=True)
    stats_ref[...] = jnp.concatenate([s1, s2], axis=1)


def _apply_kernel(x_ref, w_ref, stats_ref, g_ref, b_ref, out_ref, *,
                  kernel_size, dilation, tl, n_blk, cnt, eps):
    st = jnp.sum(stats_ref[...], axis=0)
    inv_cnt = jnp.float32(1.0 / cnt)
    mean = st[:, 0:1] * inv_cnt
    var = jnp.maximum(st[:, 1:2] * inv_cnt - mean * mean, 0.0)
    scale = g_ref[...] * jax.lax.rsqrt(var + eps)
    shift = b_ref[...] - mean * scale
    w_flat = w_ref[...]
    for b in range(n_blk):
        conv = _conv_one(x_ref[b], w_flat, kernel_size, dilation, tl)
        out_ref[b] = jnp.maximum(conv * scale + shift, 0.0)


def kernel(x, weight, bias, gamma, beta):
    del bias
    kernel_size = weight.shape[2]
    dilation = 1
    eps = 1e-3

    n, c_in, length = x.shape
    c_out = weight.shape[0]
    pad = (dilation * (kernel_size - 1)) // 2
    halo = dilation * (kernel_size - 1)
    l_out = length + 2 * pad - halo

    tl = _round_up(l_out, _LANE)
    lxp = tl + _round_up(halo, _LANE)

    x_pad = jnp.pad(
        x, ((0, 0), (0, 0), (pad, lxp - pad - length))).astype(jnp.bfloat16)
    w_flat = jnp.transpose(weight, (0, 2, 1)).reshape(
        c_out, kernel_size * c_in).astype(jnp.bfloat16)
    g2 = gamma.astype(jnp.float32).reshape(c_out, 1)
    b2 = beta.astype(jnp.float32).reshape(c_out, 1)

    n_blk = 4
    while n % n_blk:
        n_blk //= 2
    n_blocks = n // n_blk
    grid = (n_blocks,)

    x_spec = pl.BlockSpec((n_blk, c_in, lxp), lambda i: (i, 0, 0))
    w_spec = pl.BlockSpec((c_out, kernel_size * c_in), lambda i: (0, 0))
    vmem_limit = 64 * 1024 * 1024

    stats_parts = pl.pallas_call(
        functools.partial(_stats_kernel, kernel_size=kernel_size,
                          dilation=dilation, tl=tl, n_blk=n_blk,
                          l_out=l_out, c_out=c_out),
        out_shape=jax.ShapeDtypeStruct((n_blocks, c_out, 2), jnp.float32),
        grid=grid,
        in_specs=[x_spec, w_spec],
        out_specs=pl.BlockSpec((None, c_out, 2), lambda i: (i, 0, 0)),
        compiler_params=pltpu.CompilerParams(
            dimension_semantics=("parallel",),
            vmem_limit_bytes=vmem_limit),
    )(x_pad, w_flat)

    return pl.pallas_call(
        functools.partial(_apply_kernel, kernel_size=kernel_size,
                          dilation=dilation, tl=tl, n_blk=n_blk,
                          cnt=float(n * l_out), eps=eps),
        out_shape=jax.ShapeDtypeStruct((n, c_out, l_out), jnp.float32),
        grid=grid,
        in_specs=[x_spec, w_spec,
                  pl.BlockSpec((n_blocks, c_out, 2), lambda i: (0, 0, 0)),
                  pl.BlockSpec((c_out, 1), lambda i: (0, 0)),
                  pl.BlockSpec((c_out, 1), lambda i: (0, 0))],
        out_specs=pl.BlockSpec((n_blk, c_out, tl), lambda i: (i, 0, 0)),
        compiler_params=pltpu.CompilerParams(
            dimension_semantics=("parallel",),
            vmem_limit_bytes=vmem_limit),
    )(x_pad, w_flat, stats_parts, g2, b2)
